# Initial kernel scaffold; baseline (speedup 1.0000x reference)
import jax
import jax.numpy as jnp
from jax.experimental import pallas as pl


def kernel(output, mask, ind, target):
    raise NotImplementedError("write your pallas kernel here")


# speedup vs baseline: 4.2482x; 4.2482x over previous
import functools

import jax
import jax.numpy as jnp
from jax import lax
from jax.experimental import pallas as pl
from jax.experimental.pallas import tpu as pltpu
from jax.experimental.pallas import tpu_sc as plsc

B = 32
N = 100
C = 34
HW = 128 * 128
K = N * C
KP = 3456
NPAD = 128
CHUNK = 128
NCHUNK = KP // CHUNK
NVREG = KP // 16


@functools.partial(
    pl.kernel,
    out_type=jax.ShapeDtypeStruct((B, 128), jnp.float32),
    mesh=plsc.VectorSubcoreMesh(core_axis_name="c", subcore_axis_name="s"),
    compiler_params=pltpu.CompilerParams(needs_layout_passes=False),
    scratch_types=[
        pltpu.VMEM((NPAD,), jnp.int32),
        pltpu.VMEM((KP,), jnp.int32),
        pltpu.VMEM((KP,), jnp.float32),
        pltpu.VMEM((KP,), jnp.float32),
        pltpu.VMEM((KP,), jnp.int32),
        pltpu.VMEM((128,), jnp.float32),
        pltpu.SemaphoreType.DMA,
    ],
)
def _sc_loss(feat_hbm, ind_hbm, tgt_hbm, msk_hbm, out_hbm,
             ind_v, idx_v, pred_v, tgt_v, msk_v, out_v, sem):
    b = lax.axis_index("s") * 2 + lax.axis_index("c")
    lane = lax.iota(jnp.int32, 16)
    zf = jnp.zeros((16,), jnp.float32)
    cvec = jnp.full((16,), C, jnp.int32)
    nmax = jnp.full((16,), N - 1, jnp.int32)

    pltpu.sync_copy(ind_hbm.at[b], ind_v)
    pltpu.sync_copy(tgt_hbm.at[b], tgt_v)
    pltpu.sync_copy(msk_hbm.at[b], msk_v)

    bbase = b * (C * HW)

    def build(k, _):
        p = lane + k * 16
        n = lax.div(p, cvec)
        c = p - n * cvec
        n = jnp.minimum(n, nmax)
        base = plsc.load_gather(ind_v, [n])
        idx_v[pl.ds(pl.multiple_of(k * 16, 16), 16)] = bbase + c * HW + base
        return 0

    lax.fori_loop(0, NVREG, build, 0)

    def fire(m, _):
        off = pl.multiple_of(m * CHUNK, CHUNK)
        pltpu.async_copy(
            feat_hbm.at[idx_v.at[pl.ds(off, CHUNK)]],
            pred_v.at[pl.ds(off, CHUNK)],
            sem,
        )
        return 0

    lax.fori_loop(0, NCHUNK, fire, 0)
    pltpu.make_async_copy(feat_hbm.at[pl.ds(0, KP)], pred_v, sem).wait()

    def body(k, carry):
        accn, accd = carry
        o = pl.multiple_of(k * 16, 16)
        pv = pred_v[pl.ds(o, 16)]
        tv = tgt_v[pl.ds(o, 16)]
        mv = msk_v[pl.ds(o, 16)].astype(jnp.float32)
        accn = accn + jnp.abs(pv - tv) * mv
        accd = accd + mv
        return accn, accd

    accn, accd = lax.fori_loop(0, NVREG, body, (zf, zf))

    n_s = jnp.sum(accn)
    d_s = jnp.sum(accd)
    out_v[pl.ds(0, 16)] = jnp.where(lane == 0, n_s, 0.0) + jnp.where(lane == 1, d_s, 0.0)
    for off in range(16, 128, 16):
        out_v[pl.ds(off, 16)] = zf
    pltpu.sync_copy(out_v, out_hbm.at[b])


def kernel(output, mask, ind, target):
    feat = output.reshape(-1)
    ind_p = jnp.pad(ind.astype(jnp.int32), ((0, 0), (0, NPAD - N)))
    tgt_p = jnp.pad(target.reshape(B, K), ((0, 0), (0, KP - K)))
    msk_p = jnp.pad(mask.reshape(B, K), ((0, 0), (0, KP - K)))
    parts = _sc_loss(feat, ind_p, tgt_p, msk_p)
    return jnp.sum(parts[:, 0]) / (jnp.sum(parts[:, 1]) + 0.0001)


# speedup vs baseline: 4.5038x; 1.0602x over previous
import functools

import jax
import jax.numpy as jnp
from jax import lax
from jax.experimental import pallas as pl
from jax.experimental.pallas import tpu as pltpu
from jax.experimental.pallas import tpu_sc as plsc

B = 32
N = 100
C = 34
HW = 128 * 128
K = N * C
KP = 3456
NPAD = 128
CHUNK = 128
NCHUNK = KP // CHUNK
NVREG = KP // 16


@functools.partial(
    pl.kernel,
    out_type=jax.ShapeDtypeStruct((B, 128), jnp.float32),
    mesh=plsc.VectorSubcoreMesh(core_axis_name="c", subcore_axis_name="s"),
    compiler_params=pltpu.CompilerParams(needs_layout_passes=False),
    scratch_types=[
        pltpu.VMEM((NPAD,), jnp.int32),
        pltpu.VMEM((KP,), jnp.int32),
        pltpu.VMEM((KP,), jnp.float32),
        pltpu.VMEM((KP,), jnp.float32),
        pltpu.VMEM((KP,), jnp.int32),
        pltpu.VMEM((128,), jnp.float32),
        pltpu.SemaphoreType.DMA,
        pltpu.SemaphoreType.DMA,
        pltpu.SemaphoreType.DMA,
    ],
)
def _sc_loss(feat_hbm, ind_hbm, tgt_hbm, msk_hbm, out_hbm,
             ind_v, idx_v, pred_v, tgt_v, msk_v, out_v, sem, sem_ind, sem_in):
    b = lax.axis_index("s") * 2 + lax.axis_index("c")
    lane = lax.iota(jnp.int32, 16)
    zf = jnp.zeros((16,), jnp.float32)
    cvec = jnp.full((16,), C, jnp.int32)
    nmax = jnp.full((16,), N - 1, jnp.int32)

    pltpu.async_copy(ind_hbm.at[b], ind_v, sem_ind)
    pltpu.async_copy(tgt_hbm.at[b], tgt_v, sem_in)
    pltpu.async_copy(msk_hbm.at[b], msk_v, sem_in)
    pltpu.make_async_copy(ind_hbm.at[b], ind_v, sem_ind).wait()

    bbase = b * (C * HW)

    def build(k, _):
        p = lane + k * 16
        n = lax.div(p, cvec)
        c = p - n * cvec
        n = jnp.minimum(n, nmax)
        base = plsc.load_gather(ind_v, [n])
        idx_v[pl.ds(pl.multiple_of(k * 16, 16), 16)] = bbase + c * HW + base
        return 0

    lax.fori_loop(0, NVREG, build, 0, unroll=4)

    def fire(m, _):
        off = pl.multiple_of(m * CHUNK, CHUNK)
        pltpu.async_copy(
            feat_hbm.at[idx_v.at[pl.ds(off, CHUNK)]],
            pred_v.at[pl.ds(off, CHUNK)],
            sem,
        )
        return 0

    lax.fori_loop(0, NCHUNK, fire, 0)
    pltpu.make_async_copy(feat_hbm.at[pl.ds(0, KP)], pred_v, sem).wait()
    pltpu.make_async_copy(tgt_hbm.at[b], tgt_v, sem_in).wait()
    pltpu.make_async_copy(msk_hbm.at[b], msk_v, sem_in).wait()

    def body(k, carry):
        accn, accd = carry
        o = pl.multiple_of(k * 16, 16)
        pv = pred_v[pl.ds(o, 16)]
        tv = tgt_v[pl.ds(o, 16)]
        mv = msk_v[pl.ds(o, 16)].astype(jnp.float32)
        accn = accn + jnp.abs(pv - tv) * mv
        accd = accd + mv
        return accn, accd

    accn, accd = lax.fori_loop(0, NVREG, body, (zf, zf), unroll=4)

    n_s = jnp.sum(accn)
    d_s = jnp.sum(accd)
    out_v[pl.ds(0, 16)] = jnp.where(lane == 0, n_s, 0.0) + jnp.where(lane == 1, d_s, 0.0)
    for off in range(16, 128, 16):
        out_v[pl.ds(off, 16)] = zf
    pltpu.sync_copy(out_v, out_hbm.at[b])


def kernel(output, mask, ind, target):
    feat = output.reshape(-1)
    ind_p = jnp.pad(ind.astype(jnp.int32), ((0, 0), (0, NPAD - N)))
    tgt_p = jnp.pad(target.reshape(B, K), ((0, 0), (0, KP - K)))
    msk_p = jnp.pad(mask.reshape(B, K), ((0, 0), (0, KP - K)))
    parts = _sc_loss(feat, ind_p, tgt_p, msk_p)
    return jnp.sum(parts[:, 0]) / (jnp.sum(parts[:, 1]) + 0.0001)
